# Initial kernel scaffold; baseline (speedup 1.0000x reference)
import jax
import jax.numpy as jnp
from jax.experimental import pallas as pl


def kernel(sequence_outputs, idxs):
    raise NotImplementedError("write your pallas kernel here")


# speedup vs baseline: 29.9612x; 29.9612x over previous
import functools

import jax
import jax.numpy as jnp
from jax.experimental import pallas as pl
from jax.experimental.pallas import tpu as pltpu

_MAX_ANS_LEN = 30
_EPS = 1e-8
_NEG = -10000.0


def _dcr_kernel(idxs_ref, seq_ref, mv_ref, ei_ref):
    b = pl.program_id(0)
    S = seq_ref.shape[1]
    H = seq_ref.shape[2]
    sep0 = idxs_ref[b, 0]
    sep1 = idxs_ref[b, 1]

    seq = seq_ref[0]
    q1 = seq_ref[0, 1:2, :]
    q2 = seq_ref[0, pl.ds(sep0 - 1, 1), :]
    q = jnp.concatenate([q1, q2], axis=0)

    dn = (((1,), (1,)), ((), ()))
    ab = jax.lax.dot_general(q, seq, dimension_numbers=dn,
                             preferred_element_type=jnp.float32)
    n2 = jax.lax.dot_general(jnp.ones((1, H), jnp.float32), seq * seq,
                             dimension_numbers=dn,
                             preferred_element_type=jnp.float32)

    a_row = ab[0:1, :]
    b_row = ab[1:2, :]
    qn = jnp.sqrt(jnp.sum(q1 * q1) + jnp.sum(q2 * q2))
    inv_qn = 1.0 / jnp.maximum(qn, _EPS)

    pad = jnp.ones((1, _MAX_ANS_LEN + 2), jnp.float32)
    b_pad = jnp.concatenate([b_row, pad], axis=1)
    n2_pad = jnp.concatenate([n2, pad], axis=1)

    i_idx = jax.lax.broadcasted_iota(jnp.int32, (1, S), 1)

    mv = jnp.full((1, S), _NEG, jnp.float32)
    best_o = jnp.zeros((1, S), jnp.int32)
    for o in range(_MAX_ANS_LEN):
        b_o = jax.lax.slice(b_pad, (0, o), (1, o + S))
        n2_o = jax.lax.slice(n2_pad, (0, o), (1, o + S))
        num = a_row + b_o
        den = jnp.maximum(jnp.sqrt(n2 + n2_o), _EPS)
        sim = num / den * inv_qn
        valid = i_idx < (sep1 - o)
        sim = jnp.where(valid, sim, _NEG)
        if o == 0:
            mv = sim
        else:
            upd = sim > mv
            mv = jnp.where(upd, sim, mv)
            best_o = jnp.where(upd, o, best_o)

    i_valid = (i_idx > sep0) & (i_idx < sep1)
    mv_ref[0] = jnp.where(i_valid, mv, _NEG)
    ei_ref[0] = jnp.where(i_valid, i_idx + best_o, -1)


@functools.partial(jax.jit, static_argnames=())
def kernel(sequence_outputs, idxs):
    B, S, H = sequence_outputs.shape
    grid = (B,)
    out_shape = (
        jax.ShapeDtypeStruct((B, 1, S), jnp.float32),
        jax.ShapeDtypeStruct((B, 1, S), jnp.int32),
    )
    mv, ei = pl.pallas_call(
        _dcr_kernel,
        grid=grid,
        in_specs=[
            pl.BlockSpec(memory_space=pltpu.SMEM),
            pl.BlockSpec((1, S, H), lambda b: (b, 0, 0)),
        ],
        out_specs=(
            pl.BlockSpec((1, 1, S), lambda b: (b, 0, 0)),
            pl.BlockSpec((1, 1, S), lambda b: (b, 0, 0)),
        ),
        out_shape=out_shape,
        compiler_params=pltpu.CompilerParams(
            dimension_semantics=("arbitrary",),
        ),
    )(idxs, sequence_outputs)
    return mv.reshape(B, S), ei.reshape(B, S)
